# baseline (speedup 1.0000x reference)
import functools

import jax
import jax.numpy as jnp
from jax import lax
from jax.experimental import pallas as pl
from jax.experimental.pallas import tpu as pltpu
from jax.experimental.pallas import tpu_sc as plsc

_NUM_SAMPLES = 100000
_NUM_SGS = 24
_FEAT = _NUM_SGS * 7
_BATCH = 16384
_CHUNK = 128
_NTILE = _BATCH // _CHUNK
_OUT_FLAT = _BATCH * _FEAT


def _make_gather():
    info = plsc.get_sparse_core_info()
    nc, ns = info.num_cores, info.num_subcores
    nw = nc * ns
    b_per_w = _BATCH // nw
    n_chunks = b_per_w // _CHUNK
    mesh = plsc.VectorSubcoreMesh(core_axis_name="c", subcore_axis_name="s")

    @functools.partial(
        pl.kernel,
        mesh=mesh,
        compiler_params=pltpu.CompilerParams(use_tc_tiling_on_sc=False),
        out_type=jax.ShapeDtypeStruct((_OUT_FLAT,), jnp.float32),
        scratch_types=[
            pltpu.VMEM((n_chunks, _CHUNK), jnp.int32),
            pltpu.VMEM((_CHUNK, _FEAT), jnp.float32),
            pltpu.VMEM((21 * n_chunks * 8 * _CHUNK,), jnp.float32),
            pltpu.SemaphoreType.DMA,
        ],
    )
    def gather_kernel(idx_hbm, table_hbm, out_hbm, idx_v, stage, obuf, sem):
        wid = lax.axis_index("s") * nc + lax.axis_index("c")
        pltpu.sync_copy(idx_hbm.at[wid], idx_v)
        iota = lax.broadcasted_iota(jnp.int32, (16,), 0)
        masks = [(iota & s) != 0 for s in (1, 2, 4, 8)]
        perms = [iota ^ r for r in range(16)]

        for blk in range(n_chunks):
            cp = pltpu.async_copy(table_hbm.at[idx_v.at[blk]], stage, sem)
            cp.wait()

            def trans_eg(eg, carry):
                e0 = jnp.where(eg < 10, eg * 16, 152)
                for jg in range(8):
                    v = [stage[jg * 16 + r, pl.ds(e0, 16)] for r in range(16)]
                    a = [
                        v[r].at[perms[r]].get(
                            mode="promise_in_bounds", unique_indices=True
                        )
                        for r in range(16)
                    ]
                    for si, s in enumerate((1, 2, 4, 8)):
                        a = [
                            jnp.where(masks[si], a[r ^ s], a[r])
                            for r in range(16)
                        ]
                    c = [
                        a[r].at[perms[r]].get(
                            mode="promise_in_bounds", unique_indices=True
                        )
                        for r in range(16)
                    ]
                    for r in range(16):
                        e = e0 + r
                        k = lax.shift_right_logical(e * 9363, 16)
                        d = e - k * 7
                        s_ = lax.shift_right_logical(k, 3)
                        rr = lax.bitwise_and(k, 7)
                        g = d * 3 + s_
                        off = ((g * n_chunks + blk) * 8 + rr) * _CHUNK + jg * 16

                        @pl.when(
                            jnp.logical_or(eg < 10, e >= 160)
                        )
                        def _():
                            obuf[pl.ds(off, 16)] = c[r]
                return carry

            lax.fori_loop(0, 11, trans_eg, 0, unroll=False)

        c0 = wid * n_chunks
        grp = n_chunks * 8 * _CHUNK
        for g in range(21):
            pltpu.sync_copy(
                obuf.at[pl.ds(g * grp, grp)],
                out_hbm.at[pl.ds(g * _NTILE * 8 * _CHUNK + c0 * 8 * _CHUNK, grp)],
            )

    return gather_kernel


_GATHER = _make_gather()
_NW, _NCHUNKS = 32, 4


def kernel(idxs, sgs):
    idx3 = idxs.astype(jnp.int32).reshape(_NW, _NCHUNKS, _CHUNK)
    table = sgs.reshape(_NUM_SAMPLES, _FEAT)
    o = _GATHER(idx3, table)
    o = o.reshape(7, 3, _NTILE, 8, _CHUNK)
    return jnp.transpose(o, (2, 4, 1, 3, 0)).reshape(_BATCH, _NUM_SGS, 7)
